# baseline (speedup 1.0000x reference)
import jax
import jax.numpy as jnp
from jax import lax
from jax.experimental import pallas as pl
from jax.experimental.pallas import tpu as pltpu
from jax.experimental.pallas import tpu_sc as plsc

E = 320000
D = 128
R = 1000
RPAD = 1024
DUMMY = 1000
NC = 2
NS = 16
NW = NC * NS
EW = E // NW
C = 80
NCHUNK = EW // C
NG = C // 8
NL = D // 16


def _sc_body(h_hbm, et_hbm, out_hbm, h_v, et_v, lout, segid, slot_m, acc_sh,
             sem0, sem1):
    cid = lax.axis_index("c")
    sid = lax.axis_index("s")
    wid = cid * NS + sid
    iota = lax.iota(jnp.int32, 16)
    zero16 = jnp.zeros((16,), jnp.float32)
    dummy16 = jnp.full((16,), DUMMY, jnp.int32)

    def _z(i, _):
        for k in range(NL):
            lout[i, pl.ds(16 * k, 16)] = zero16
        return 0
    lax.fori_loop(0, C, _z, 0)

    @pl.when(sid == 0)
    def _():
        for b in range(RPAD // C):
            pltpu.sync_copy(lout, acc_sh.at[pl.ds(b * C, C)])
        rem = RPAD - (RPAD // C) * C
        pltpu.sync_copy(lout.at[pl.ds(0, rem)],
                        acc_sh.at[pl.ds((RPAD // C) * C, rem)])
    plsc.subcore_barrier()

    et_v[0, pl.ds(0, 16)] = jnp.full((16,), -1, jnp.int32)
    et_v[1, pl.ds(0, 16)] = jnp.full((16,), -1, jnp.int32)

    def _start(ci, b, sem):
        base = wid * EW + ci * C
        pltpu.async_copy(et_hbm.at[pl.ds(base, C)],
                         et_v.at[b, pl.ds(8, C)], sem)
        pltpu.async_copy(h_hbm.at[pl.ds(base, C)], h_v.at[b], sem)

    def _wait(ci, b, sem):
        base = wid * EW + ci * C
        pltpu.make_async_copy(et_hbm.at[pl.ds(base, C)],
                              et_v.at[b, pl.ds(8, C)], sem).wait()
        pltpu.make_async_copy(h_hbm.at[pl.ds(base, C)], h_v.at[b], sem).wait()

    lane_idx = [jnp.full((16,), l, jnp.int32) for l in range(16)]
    col_idx = [lax.iota(jnp.int32, 16) + 16 * k for k in range(NL)]

    def _process(b):
        for t in range(C // 16):
            segid[pl.ds(16 * t, 16)] = dummy16

        def _group(gi, carry):
            blp = carry[0]
            acc = list(carry[1:])
            i0 = gi * 16
            ev = et_v[b, pl.ds(8 + i0, 16)]
            evm1 = et_v[b, pl.ds(7 + i0, 16)]
            isnew = (ev != evm1).astype(jnp.int32)
            slot = plsc.cumsum(isnew) + blp
            slot_m[pl.ds(0, 16)] = slot
            plsc.store_scatter(segid, [slot], ev)
            for l in range(16):
                bl = plsc.load_gather(slot_m, [lane_idx[l]])
                opens = bl != blp
                for k in range(NL):
                    hk = h_v[b, i0 + l, pl.ds(16 * k, 16)]
                    ak = jnp.where(opens, hk, acc[k] + hk)
                    plsc.store_scatter(lout, [bl, col_idx[k]], ak)
                    acc[k] = ak
                blp = bl
            return (blp,) + tuple(acc)

        init = (jnp.full((16,), -1, jnp.int32),) + tuple(
            zero16 for _ in range(NL))
        fin = lax.fori_loop(0, C // 16, _group, init)
        ng = (fin[0][0] + 16) >> 4

        def _scat(gi, _):
            idxv = segid[pl.ds(gi * 16, 16)]
            pltpu.sync_copy(lout.at[pl.ds(gi * 16, 16)],
                            acc_sh.at[idxv], add=True)
            return 0
        lax.fori_loop(0, ng, _scat, 0)

    _start(0, 0, sem0)

    def _pair(ci, _):
        c0 = ci * 2
        _start(c0 + 1, 1, sem1)
        _wait(c0, 0, sem0)
        _process(0)
        _start(c0 + 2, 0, sem0)
        _wait(c0 + 1, 1, sem1)
        _process(1)
        return 0
    lax.fori_loop(0, NCHUNK // 2, _pair, 0)
    _wait(NCHUNK - 1, 0, sem0)
    _process(0)

    plsc.subcore_barrier()
    rows_pt = RPAD // NS
    r0 = sid * rows_pt
    pltpu.sync_copy(acc_sh.at[pl.ds(r0, rows_pt)], lout.at[pl.ds(0, rows_pt)])
    pltpu.sync_copy(lout.at[pl.ds(0, rows_pt)],
                    out_hbm.at[cid, pl.ds(r0, rows_pt)])


def _finish_kernel(p_ref, g_ref, wr_ref, w1_ref, br_ref, b1_ref, out_ref):
    g_edges = p_ref[0, :R, :] + p_ref[1, :R, :]
    norm = jnp.sqrt(jnp.sum(g_edges * g_edges, axis=1, keepdims=True))
    g_edges = g_edges / jnp.maximum(norm, 1e-12)
    g_edges = jnp.where(g_edges > 0, g_edges, jnp.exp(g_edges) - 1.0)
    t1 = lax.dot_general(g_ref[...], wr_ref[...], (((1,), (1,)), ((), ())),
                         preferred_element_type=jnp.float32)
    t2 = lax.dot_general(g_edges, w1_ref[...], (((1,), (1,)), ((), ())),
                         preferred_element_type=jnp.float32)
    out_ref[...] = t1 + t2 + br_ref[...][None, :] + b1_ref[...][None, :]


def kernel(h_ijk, g, edge_type, Wr, br, W1, b1):
    et = jnp.asarray(edge_type, jnp.int32)

    seg_fn = pl.kernel(
        _sc_body,
        out_type=jax.ShapeDtypeStruct((NC, RPAD, D), jnp.float32),
        mesh=plsc.VectorSubcoreMesh(core_axis_name="c", subcore_axis_name="s"),
        compiler_params=pltpu.CompilerParams(needs_layout_passes=False),
        scratch_types=[
            pltpu.VMEM((2, C, D), jnp.float32),
            pltpu.VMEM((2, 8 + C), jnp.int32),
            pltpu.VMEM((C, D), jnp.float32),
            pltpu.VMEM((C,), jnp.int32),
            pltpu.VMEM((16,), jnp.int32),
            pltpu.VMEM_SHARED((RPAD, D), jnp.float32),
            pltpu.SemaphoreType.DMA,
            pltpu.SemaphoreType.DMA,
        ],
    )
    partial = seg_fn(h_ijk, et)

    return pl.pallas_call(
        _finish_kernel,
        out_shape=jax.ShapeDtypeStruct((R, 64), jnp.float32),
    )(partial, g, Wr, W1, br, b1)


# speedup vs baseline: 3.4245x; 3.4245x over previous
import jax
import jax.numpy as jnp
from jax import lax
from jax.experimental import pallas as pl
from jax.experimental.pallas import tpu as pltpu
from jax.experimental.pallas import tpu_sc as plsc

E = 320000
D = 128
R = 1000
RPAD = 1024
DUMMY = 1000
NC = 2
NS = 16
NW = NC * NS
EW = E // NW
C = 80
NCHUNK = EW // C
NG = C // 8
NL = D // 16


def _sc_body(h_hbm, et_hbm, out_hbm, h_v, et_v, lout, segid, slot_m, acc_sh,
             sem0, sem1):
    cid = lax.axis_index("c")
    sid = lax.axis_index("s")
    wid = cid * NS + sid
    iota = lax.iota(jnp.int32, 16)
    zero16 = jnp.zeros((16,), jnp.float32)
    dummy16 = jnp.full((16,), DUMMY, jnp.int32)

    def _z(i, _):
        for k in range(NL):
            lout[i, pl.ds(16 * k, 16)] = zero16
        return 0
    lax.fori_loop(0, C, _z, 0)

    @pl.when(sid == 0)
    def _():
        for b in range(RPAD // C):
            pltpu.sync_copy(lout, acc_sh.at[pl.ds(b * C, C)])
        rem = RPAD - (RPAD // C) * C
        pltpu.sync_copy(lout.at[pl.ds(0, rem)],
                        acc_sh.at[pl.ds((RPAD // C) * C, rem)])
    plsc.subcore_barrier()

    et_v[0, pl.ds(0, 16)] = jnp.full((16,), -1, jnp.int32)
    et_v[1, pl.ds(0, 16)] = jnp.full((16,), -1, jnp.int32)

    def _start(ci, b, sem):
        base = wid * EW + ci * C
        pltpu.async_copy(et_hbm.at[pl.ds(base, C)],
                         et_v.at[b, pl.ds(8, C)], sem)
        pltpu.async_copy(h_hbm.at[pl.ds(base, C)], h_v.at[b, pl.ds(0, C)],
                         sem)

    def _wait(ci, b, sem):
        base = wid * EW + ci * C
        pltpu.make_async_copy(et_hbm.at[pl.ds(base, C)],
                              et_v.at[b, pl.ds(8, C)], sem).wait()
        pltpu.make_async_copy(h_hbm.at[pl.ds(base, C)],
                              h_v.at[b, pl.ds(0, C)], sem).wait()

    lane_idx = [jnp.full((16,), 16 + l, jnp.int32) for l in range(16)]
    col_idx = [lax.iota(jnp.int32, 16) + 16 * k for k in range(NL)]

    def _process(b):
        for t in range(C // 16):
            segid[pl.ds(16 * t, 16)] = dummy16

        def _group(gi, carry):
            blp = carry[0]
            acc = list(carry[1:1 + NL])
            hc = list(carry[1 + NL:])
            i0 = gi * 16
            ev = et_v[b, pl.ds(8 + i0, 16)]
            evm1 = et_v[b, pl.ds(7 + i0, 16)]
            isnew = (ev != evm1).astype(jnp.int32)
            slot = plsc.cumsum(isnew) + blp
            slot_m[pl.ds(16, 16)] = slot
            plsc.store_scatter(segid, [slot], ev)
            bl = plsc.load_gather(slot_m, [lane_idx[0]])
            for l in range(16):
                vm = bl != blp
                bln = plsc.load_gather(slot_m, [lane_idx[min(l + 1, 15)]])
                a = [None] * NL
                hn = [None] * NL
                for k in range(NL):
                    hn[k] = h_v[b, i0 + l + 1, pl.ds(16 * k, 16)]
                    a[k] = hc[k] + jnp.where(vm, zero16, acc[k])
                    if k >= 2:
                        plsc.store_scatter(lout, [bl, col_idx[k - 2]],
                                           a[k - 2])
                plsc.store_scatter(lout, [bl, col_idx[NL - 2]], a[NL - 2])
                plsc.store_scatter(lout, [bl, col_idx[NL - 1]], a[NL - 1])
                acc = a
                hc = hn
                blp = bl
                bl = bln
            return (blp,) + tuple(acc) + tuple(hc)

        hc0 = [h_v[b, 0, pl.ds(16 * k, 16)] for k in range(NL)]
        neg1v = jnp.full((16,), -1, jnp.int32)
        init = (neg1v,) + tuple(zero16 for _ in range(NL)) + tuple(hc0)
        fin = lax.fori_loop(0, C // 16, _group, init)
        ng = (fin[0][0] + 16) >> 4

        def _scat(gi, _):
            idxv = segid[pl.ds(gi * 16, 16)]
            pltpu.sync_copy(lout.at[pl.ds(gi * 16, 16)],
                            acc_sh.at[idxv], add=True)
            return 0
        lax.fori_loop(0, ng, _scat, 0)

    _start(0, 0, sem0)

    def _pair(ci, _):
        c0 = ci * 2
        _start(c0 + 1, 1, sem1)
        _wait(c0, 0, sem0)
        _process(0)
        _start(c0 + 2, 0, sem0)
        _wait(c0 + 1, 1, sem1)
        _process(1)
        return 0
    lax.fori_loop(0, NCHUNK // 2, _pair, 0)
    _wait(NCHUNK - 1, 0, sem0)
    _process(0)

    plsc.subcore_barrier()
    rows_pt = RPAD // NS
    r0 = sid * rows_pt
    pltpu.sync_copy(acc_sh.at[pl.ds(r0, rows_pt)], lout.at[pl.ds(0, rows_pt)])
    pltpu.sync_copy(lout.at[pl.ds(0, rows_pt)],
                    out_hbm.at[cid, pl.ds(r0, rows_pt)])


def _finish_kernel(p_ref, g_ref, wr_ref, w1_ref, br_ref, b1_ref, out_ref):
    g_edges = p_ref[0, :R, :] + p_ref[1, :R, :]
    norm = jnp.sqrt(jnp.sum(g_edges * g_edges, axis=1, keepdims=True))
    g_edges = g_edges / jnp.maximum(norm, 1e-12)
    g_edges = jnp.where(g_edges > 0, g_edges, jnp.exp(g_edges) - 1.0)
    t1 = lax.dot_general(g_ref[...], wr_ref[...], (((1,), (1,)), ((), ())),
                         preferred_element_type=jnp.float32)
    t2 = lax.dot_general(g_edges, w1_ref[...], (((1,), (1,)), ((), ())),
                         preferred_element_type=jnp.float32)
    out_ref[...] = t1 + t2 + br_ref[...][None, :] + b1_ref[...][None, :]


def kernel(h_ijk, g, edge_type, Wr, br, W1, b1):
    et = jnp.asarray(edge_type, jnp.int32)

    seg_fn = pl.kernel(
        _sc_body,
        out_type=jax.ShapeDtypeStruct((NC, RPAD, D), jnp.float32),
        mesh=plsc.VectorSubcoreMesh(core_axis_name="c", subcore_axis_name="s"),
        compiler_params=pltpu.CompilerParams(needs_layout_passes=False),
        scratch_types=[
            pltpu.VMEM((2, C + 8, D), jnp.float32),
            pltpu.VMEM((2, 8 + C), jnp.int32),
            pltpu.VMEM((C, D), jnp.float32),
            pltpu.VMEM((C,), jnp.int32),
            pltpu.VMEM((32,), jnp.int32),
            pltpu.VMEM_SHARED((RPAD, D), jnp.float32),
            pltpu.SemaphoreType.DMA,
            pltpu.SemaphoreType.DMA,
        ],
    )
    partial = seg_fn(h_ijk, et)

    return pl.pallas_call(
        _finish_kernel,
        out_shape=jax.ShapeDtypeStruct((R, 64), jnp.float32),
    )(partial, g, Wr, W1, br, b1)


# speedup vs baseline: 3.5681x; 1.0419x over previous
import jax
import jax.numpy as jnp
from jax import lax
from jax.experimental import pallas as pl
from jax.experimental.pallas import tpu as pltpu
from jax.experimental.pallas import tpu_sc as plsc

E = 320000
D = 128
R = 1000
RPAD = 1024
DUMMY = 1000
NC = 2
NS = 16
NW = NC * NS
EW = E // NW
C = 80
NCHUNK = EW // C
NG = C // 8
NL = D // 16


def _sc_body(h_hbm, et_hbm, out_hbm, h_v, et_v, lout, segid, slot_m, acc_sh,
             sem0, sem1):
    cid = lax.axis_index("c")
    sid = lax.axis_index("s")
    wid = cid * NS + sid
    iota = lax.iota(jnp.int32, 16)
    zero16 = jnp.zeros((16,), jnp.float32)
    dummy16 = jnp.full((16,), DUMMY, jnp.int32)

    def _z(i, _):
        for k in range(NL):
            lout[i, pl.ds(16 * k, 16)] = zero16
        return 0
    lax.fori_loop(0, C, _z, 0)

    @pl.when(sid == 0)
    def _():
        for b in range(RPAD // C):
            pltpu.sync_copy(lout, acc_sh.at[pl.ds(b * C, C)])
        rem = RPAD - (RPAD // C) * C
        pltpu.sync_copy(lout.at[pl.ds(0, rem)],
                        acc_sh.at[pl.ds((RPAD // C) * C, rem)])
    plsc.subcore_barrier()

    et_v[0, pl.ds(0, 16)] = jnp.full((16,), -1, jnp.int32)
    et_v[1, pl.ds(0, 16)] = jnp.full((16,), -1, jnp.int32)

    def _start(ci, b, sem):
        base = wid * EW + ci * C
        pltpu.async_copy(et_hbm.at[pl.ds(base, C)],
                         et_v.at[b, pl.ds(8, C)], sem)
        pltpu.async_copy(h_hbm.at[pl.ds(base, C)], h_v.at[b, pl.ds(0, C)],
                         sem)

    def _wait(ci, b, sem):
        base = wid * EW + ci * C
        pltpu.make_async_copy(et_hbm.at[pl.ds(base, C)],
                              et_v.at[b, pl.ds(8, C)], sem).wait()
        pltpu.make_async_copy(h_hbm.at[pl.ds(base, C)],
                              h_v.at[b, pl.ds(0, C)], sem).wait()

    lane_idx = [jnp.full((16,), 16 + l, jnp.int32) for l in range(16)]
    col_idx = [lax.iota(jnp.int32, 16) + 16 * k for k in range(NL)]

    def _process(b):
        for t in range(C // 16):
            segid[pl.ds(16 * t, 16)] = dummy16

        def _group(gi, carry):
            blp = carry[0]
            acc = list(carry[1:1 + NL])
            hc = list(carry[1 + NL:])
            i0 = gi * 16
            ev = et_v[b, pl.ds(8 + i0, 16)]
            evm1 = et_v[b, pl.ds(7 + i0, 16)]
            isnew = (ev != evm1).astype(jnp.int32)
            slot = plsc.cumsum(isnew) + blp
            slot_m[pl.ds(16, 16)] = slot
            plsc.store_scatter(segid, [slot], ev)
            bl = plsc.load_gather(slot_m, [lane_idx[0]])
            for l in range(16):
                vm = bl != blp
                bln = plsc.load_gather(slot_m, [lane_idx[min(l + 1, 15)]])
                a = [None] * NL
                hn = [None] * NL
                for k in range(NL):
                    hn[k] = h_v[b, i0 + l + 1, pl.ds(16 * k, 16)]
                    a[k] = hc[k] + jnp.where(vm, zero16, acc[k])
                    if k >= 2:
                        plsc.store_scatter(lout, [bl, col_idx[k - 2]],
                                           a[k - 2])
                plsc.store_scatter(lout, [bl, col_idx[NL - 2]], a[NL - 2])
                plsc.store_scatter(lout, [bl, col_idx[NL - 1]], a[NL - 1])
                acc = a
                hc = hn
                blp = bl
                bl = bln
            return (blp,) + tuple(acc) + tuple(hc)

        hc0 = [h_v[b, 0, pl.ds(16 * k, 16)] for k in range(NL)]
        neg1v = jnp.full((16,), -1, jnp.int32)
        init = (neg1v,) + tuple(zero16 for _ in range(NL)) + tuple(hc0)
        if False:
            fin = lax.fori_loop(0, C // 16, _group, init)
            ng = (fin[0][0] + 16) >> 4

            def _scat(gi, _):
                idxv = segid[pl.ds(gi * 16, 16)]
                pltpu.sync_copy(lout.at[pl.ds(gi * 16, 16)],
                                acc_sh.at[idxv], add=True)
                return 0
            lax.fori_loop(0, ng, _scat, 0)
        for gi in range(C // 16):
            idxv = et_v[b, pl.ds(8 + gi * 16, 16)]
            pltpu.sync_copy(h_v.at[b, pl.ds(gi * 16, 16)],
                            acc_sh.at[idxv], add=True)

    _start(0, 0, sem0)

    def _pair(ci, _):
        c0 = ci * 2
        _start(c0 + 1, 1, sem1)
        _wait(c0, 0, sem0)
        _process(0)
        _start(c0 + 2, 0, sem0)
        _wait(c0 + 1, 1, sem1)
        _process(1)
        return 0
    lax.fori_loop(0, NCHUNK // 2, _pair, 0)
    _wait(NCHUNK - 1, 0, sem0)
    _process(0)

    plsc.subcore_barrier()
    rows_pt = RPAD // NS
    r0 = sid * rows_pt
    pltpu.sync_copy(acc_sh.at[pl.ds(r0, rows_pt)], lout.at[pl.ds(0, rows_pt)])
    pltpu.sync_copy(lout.at[pl.ds(0, rows_pt)],
                    out_hbm.at[cid, pl.ds(r0, rows_pt)])


def _finish_kernel(p_ref, g_ref, wr_ref, w1_ref, br_ref, b1_ref, out_ref):
    g_edges = p_ref[0, :R, :] + p_ref[1, :R, :]
    norm = jnp.sqrt(jnp.sum(g_edges * g_edges, axis=1, keepdims=True))
    g_edges = g_edges / jnp.maximum(norm, 1e-12)
    g_edges = jnp.where(g_edges > 0, g_edges, jnp.exp(g_edges) - 1.0)
    t1 = lax.dot_general(g_ref[...], wr_ref[...], (((1,), (1,)), ((), ())),
                         preferred_element_type=jnp.float32)
    t2 = lax.dot_general(g_edges, w1_ref[...], (((1,), (1,)), ((), ())),
                         preferred_element_type=jnp.float32)
    out_ref[...] = t1 + t2 + br_ref[...][None, :] + b1_ref[...][None, :]


def kernel(h_ijk, g, edge_type, Wr, br, W1, b1):
    et = jnp.asarray(edge_type, jnp.int32)

    seg_fn = pl.kernel(
        _sc_body,
        out_type=jax.ShapeDtypeStruct((NC, RPAD, D), jnp.float32),
        mesh=plsc.VectorSubcoreMesh(core_axis_name="c", subcore_axis_name="s"),
        compiler_params=pltpu.CompilerParams(needs_layout_passes=False),
        scratch_types=[
            pltpu.VMEM((2, C + 8, D), jnp.float32),
            pltpu.VMEM((2, 8 + C), jnp.int32),
            pltpu.VMEM((C, D), jnp.float32),
            pltpu.VMEM((C,), jnp.int32),
            pltpu.VMEM((32,), jnp.int32),
            pltpu.VMEM_SHARED((RPAD, D), jnp.float32),
            pltpu.SemaphoreType.DMA,
            pltpu.SemaphoreType.DMA,
        ],
    )
    partial = seg_fn(h_ijk, et)

    return pl.pallas_call(
        _finish_kernel,
        out_shape=jax.ShapeDtypeStruct((R, 64), jnp.float32),
    )(partial, g, Wr, W1, br, b1)


# speedup vs baseline: 3.8506x; 1.0792x over previous
import jax
import jax.numpy as jnp
from jax import lax
from jax.experimental import pallas as pl
from jax.experimental.pallas import tpu as pltpu
from jax.experimental.pallas import tpu_sc as plsc

E = 320000
D = 128
R = 1000
RPAD = 1024
NC = 2
NS = 16
NW = NC * NS
EW = E // NW
C = 80
NCHUNK = EW // C
NBUF = 4
NG = C // 16
NL = D // 16
ZROWS = 64


def _sc_body(h_hbm, et_hbm, out_hbm, h_v, et_all, zbuf, acc_sh,
             sh0, sh1, sh2, sh3, ss0, ss1, ss2, ss3):
    cid = lax.axis_index("c")
    sid = lax.axis_index("s")
    wid = cid * NS + sid
    e0 = wid * EW
    semh = [sh0, sh1, sh2, sh3]
    sems = [ss0, ss1, ss2, ss3]
    zero16 = jnp.zeros((16,), jnp.float32)

    def _z(i, _):
        for k in range(NL):
            zbuf[i, pl.ds(16 * k, 16)] = zero16
        return 0
    lax.fori_loop(0, ZROWS, _z, 0)

    @pl.when(sid == 0)
    def _():
        for t in range(RPAD // ZROWS):
            pltpu.sync_copy(zbuf, acc_sh.at[pl.ds(t * ZROWS, ZROWS)])
    pltpu.sync_copy(et_hbm.at[pl.ds(e0, EW)], et_all)
    plsc.subcore_barrier()

    def _start_h(ci, b):
        pltpu.async_copy(h_hbm.at[pl.ds(e0 + ci * C, C)], h_v.at[b], semh[b])

    def _wait_h(ci, b):
        pltpu.make_async_copy(h_hbm.at[pl.ds(e0 + ci * C, C)], h_v.at[b],
                              semh[b]).wait()

    def _fire_s(ci, b):
        for gi in range(NG):
            idxv = et_all[pl.ds(ci * C + gi * 16, 16)]
            pltpu.async_copy(h_v.at[b, pl.ds(gi * 16, 16)],
                             acc_sh.at[idxv], sems[b], add=True)

    def _drain_s(b):
        for gi in range(NG):
            pltpu.make_async_copy(h_hbm.at[pl.ds(0, 16)],
                                  h_v.at[b, pl.ds(gi * 16, 16)],
                                  sems[b]).wait()

    _start_h(0, 0)
    _start_h(1, 1)

    def _quad(i, _):
        for j in range(NBUF):
            c = i * NBUF + j

            _wait_h(c, j)
            _fire_s(c, j)

            @pl.when(c >= 2)
            def _():
                _drain_s((j + 2) % NBUF)

            @pl.when(c + 2 < NCHUNK)
            def _():
                _start_h(c + 2, (j + 2) % NBUF)
        return 0
    lax.fori_loop(0, NCHUNK // NBUF, _quad, 0)

    ct = NCHUNK - 1
    _wait_h(ct, ct % NBUF)
    _fire_s(ct, ct % NBUF)
    _drain_s((ct - 2) % NBUF)
    _drain_s((ct - 1) % NBUF)
    _drain_s(ct % NBUF)

    plsc.subcore_barrier()
    rows_pt = RPAD // NS
    r0 = sid * rows_pt
    pltpu.sync_copy(acc_sh.at[pl.ds(r0, rows_pt)],
                    h_v.at[0, pl.ds(0, rows_pt)])
    pltpu.sync_copy(h_v.at[0, pl.ds(0, rows_pt)],
                    out_hbm.at[cid, pl.ds(r0, rows_pt)])


def _finish_kernel(p_ref, g_ref, wr_ref, w1_ref, br_ref, b1_ref, out_ref):
    g_edges = p_ref[0, :R, :] + p_ref[1, :R, :]
    norm = jnp.sqrt(jnp.sum(g_edges * g_edges, axis=1, keepdims=True))
    g_edges = g_edges / jnp.maximum(norm, 1e-12)
    g_edges = jnp.where(g_edges > 0, g_edges, jnp.exp(g_edges) - 1.0)
    t1 = lax.dot_general(g_ref[...], wr_ref[...], (((1,), (1,)), ((), ())),
                         preferred_element_type=jnp.float32)
    t2 = lax.dot_general(g_edges, w1_ref[...], (((1,), (1,)), ((), ())),
                         preferred_element_type=jnp.float32)
    out_ref[...] = t1 + t2 + br_ref[...][None, :] + b1_ref[...][None, :]


def kernel(h_ijk, g, edge_type, Wr, br, W1, b1):
    et = jnp.asarray(edge_type, jnp.int32)

    seg_fn = pl.kernel(
        _sc_body,
        out_type=jax.ShapeDtypeStruct((NC, RPAD, D), jnp.float32),
        mesh=plsc.VectorSubcoreMesh(core_axis_name="c", subcore_axis_name="s"),
        compiler_params=pltpu.CompilerParams(needs_layout_passes=False),
        scratch_types=(
            [
                pltpu.VMEM((NBUF, C, D), jnp.float32),
                pltpu.VMEM((EW,), jnp.int32),
                pltpu.VMEM((ZROWS, D), jnp.float32),
                pltpu.VMEM_SHARED((RPAD, D), jnp.float32),
            ]
            + [pltpu.SemaphoreType.DMA] * (2 * NBUF)
        ),
    )
    partial = seg_fn(h_ijk, et)

    return pl.pallas_call(
        _finish_kernel,
        out_shape=jax.ShapeDtypeStruct((R, 64), jnp.float32),
    )(partial, g, Wr, W1, br, b1)
